# baseline (speedup 1.0000x reference)
import jax
import jax.numpy as jnp
from jax import lax
from jax.experimental import pallas as pl
from jax.experimental.pallas import tpu as pltpu

ROWS = 128
COLS = 32768
BC = 2048
GRID = COLS // BC
BIG = 2**31 - 1


def _body(in_ref, out_ref, max_ref, idx_ref):
    i = pl.program_id(0)
    x = in_ref[...]
    bmax = jnp.max(x, axis=1, keepdims=True)
    colid = lax.broadcasted_iota(jnp.int32, (ROWS, BC), 1)
    bidx = jnp.min(
        jnp.where(x == bmax, colid, BIG), axis=1, keepdims=True
    ) + i * BC

    @pl.when(i == 0)
    def _():
        max_ref[...] = bmax
        idx_ref[...] = bidx

    @pl.when(i > 0)
    def _():
        upd = bmax > max_ref[...]
        max_ref[...] = jnp.where(upd, bmax, max_ref[...])
        idx_ref[...] = jnp.where(upd, bidx, idx_ref[...])

    @pl.when(i == GRID - 1)
    def _():
        out_ref[...] = idx_ref[...]


def kernel(inputs):
    out = pl.pallas_call(
        _body,
        grid=(GRID,),
        in_specs=[pl.BlockSpec((ROWS, BC), lambda i: (0, i))],
        out_specs=pl.BlockSpec((ROWS, 1), lambda i: (0, 0)),
        out_shape=jax.ShapeDtypeStruct((ROWS, 1), jnp.int32),
        scratch_shapes=[
            pltpu.VMEM((ROWS, 1), jnp.float32),
            pltpu.VMEM((ROWS, 1), jnp.int32),
        ],
    )(inputs)
    return out.reshape(ROWS)


# speedup vs baseline: 1.6993x; 1.6993x over previous
import jax
import jax.numpy as jnp
from jax import lax
from jax.experimental import pallas as pl
from jax.experimental.pallas import tpu as pltpu

ROWS = 128
COLS = 32768
BC = 8192
GRID = COLS // BC
BIG = 2**31 - 1


def _body(in_ref, out_ref, max_ref, idx_ref):
    i = pl.program_id(0)
    x = in_ref[...]
    bmax = jnp.max(x, axis=1, keepdims=True)
    colid = lax.broadcasted_iota(jnp.int32, (ROWS, BC), 1)
    bidx = jnp.min(
        jnp.where(x == bmax, colid, BIG), axis=1, keepdims=True
    ) + i * BC

    @pl.when(i == 0)
    def _():
        max_ref[...] = bmax
        idx_ref[...] = bidx

    @pl.when(i > 0)
    def _():
        upd = bmax > max_ref[...]
        max_ref[...] = jnp.where(upd, bmax, max_ref[...])
        idx_ref[...] = jnp.where(upd, bidx, idx_ref[...])

    @pl.when(i == GRID - 1)
    def _():
        out_ref[...] = idx_ref[...]


def kernel(inputs):
    out = pl.pallas_call(
        _body,
        grid=(GRID,),
        in_specs=[pl.BlockSpec((ROWS, BC), lambda i: (0, i))],
        out_specs=pl.BlockSpec((ROWS, 1), lambda i: (0, 0)),
        out_shape=jax.ShapeDtypeStruct((ROWS, 1), jnp.int32),
        scratch_shapes=[
            pltpu.VMEM((ROWS, 1), jnp.float32),
            pltpu.VMEM((ROWS, 1), jnp.int32),
        ],
    )(inputs)
    return out.reshape(ROWS)


# speedup vs baseline: 2.0510x; 1.2070x over previous
import jax
import jax.numpy as jnp
from jax import lax
from jax.experimental import pallas as pl
from jax.experimental.pallas import tpu as pltpu

ROWS = 128
COLS = 32768
BC = 16384
GRID = COLS // BC
BIG = float(2**30)


def _body(in_ref, out_ref, max_ref, idx_ref):
    i = pl.program_id(0)
    x = in_ref[...]
    bmax = jnp.max(x, axis=1, keepdims=True)
    colid = lax.broadcasted_iota(jnp.int32, (ROWS, BC), 1)
    bidx = jnp.min(
        jnp.where(x == bmax, colid, 2**30), axis=1, keepdims=True
    ) + i * BC

    @pl.when(i == 0)
    def _():
        max_ref[...] = bmax
        idx_ref[...] = bidx

    @pl.when(i > 0)
    def _():
        upd = bmax > max_ref[...]
        max_ref[...] = jnp.where(upd, bmax, max_ref[...])
        idx_ref[...] = jnp.where(upd, bidx, idx_ref[...])

    @pl.when(i == GRID - 1)
    def _():
        idx_f = idx_ref[...].astype(jnp.float32)
        out_ref[...] = jnp.transpose(idx_f).astype(jnp.int32)


def kernel(inputs):
    out = pl.pallas_call(
        _body,
        grid=(GRID,),
        in_specs=[pl.BlockSpec((ROWS, BC), lambda i: (0, i))],
        out_specs=pl.BlockSpec((1, ROWS), lambda i: (0, 0)),
        out_shape=jax.ShapeDtypeStruct((1, ROWS), jnp.int32),
        scratch_shapes=[
            pltpu.VMEM((ROWS, 1), jnp.float32),
            pltpu.VMEM((ROWS, 1), jnp.int32),
        ],
    )(inputs)
    return out.reshape(ROWS)
